# Initial kernel scaffold; baseline (speedup 1.0000x reference)
import jax
import jax.numpy as jnp
from jax.experimental import pallas as pl


def kernel(positions, embedding):
    raise NotImplementedError("write your pallas kernel here")


# speedup vs baseline: 5.1416x; 5.1416x over previous
import functools

import jax
import jax.numpy as jnp
from jax import lax
from jax.experimental import pallas as pl
from jax.experimental.pallas import tpu as pltpu
from jax.experimental.pallas import tpu_sc as plsc

MAX_POS = 2048
DEPTH = 4
ROWS, COLS = 16384, 200
N = ROWS * COLS
NC, NS, L = 2, 16, 16
NW = NC * NS
PER_W = N // NW
CHUNK = 5120
NCHUNK = PER_W // CHUNK
GROUPS = CHUNK // L
UNROLL = 4
TBL = (MAX_POS + 1) * DEPTH
TBL_PAD = 8208

_mesh = plsc.VectorSubcoreMesh(core_axis_name="c", subcore_axis_name="s")


@functools.partial(
    pl.kernel,
    out_type=jax.ShapeDtypeStruct((N * DEPTH,), jnp.float32),
    mesh=_mesh,
    scratch_types=[
        pltpu.VMEM((TBL_PAD,), jnp.float32),
        pltpu.VMEM((CHUNK,), jnp.int32),
        pltpu.VMEM((CHUNK * DEPTH,), jnp.float32),
    ],
    compiler_params=pltpu.CompilerParams(needs_layout_passes=False),
)
def _lookup(pos_hbm, tbl_hbm, out_hbm, tbl_v, idx_v, out_v):
    wid = lax.axis_index("s") * NC + lax.axis_index("c")
    base_w = wid * PER_W

    pltpu.sync_copy(tbl_hbm, tbl_v)

    lane = lax.iota(jnp.int32, L)
    lane_div = lane >> 2
    lane_mod = lane & 3

    _gdn = lax.GatherDimensionNumbers(
        offset_dims=(), collapsed_slice_dims=(0,), start_index_map=(0,))

    def _vperm(x, idx):
        return lax.gather(x, idx[:, None], dimension_numbers=_gdn,
                          slice_sizes=(1,),
                          mode=lax.GatherScatterMode.PROMISE_IN_BOUNDS)

    def chunk_body(c, _):
        base = base_w + c * CHUNK
        pltpu.sync_copy(pos_hbm.at[pl.ds(base, CHUNK)], idx_v)

        def group_body(t, _):
            for u in range(UNROLL):
                g = t * UNROLL + u
                v = idx_v[pl.ds(g * L, L)]
                f = jnp.minimum(v, MAX_POS) * DEPTH
                obase = g * (L * DEPTH)
                for k in range(DEPTH):
                    flat = _vperm(f, lane_div + 4 * k) + lane_mod
                    gk = plsc.load_gather(tbl_v, [flat])
                    out_v[pl.ds(obase + k * L, L)] = gk
            return 0

        lax.fori_loop(0, GROUPS // UNROLL, group_body, 0)
        pltpu.sync_copy(out_v, out_hbm.at[pl.ds(base * DEPTH, CHUNK * DEPTH)])
        return 0

    lax.fori_loop(0, NCHUNK, chunk_body, 0)


def kernel(positions, embedding):
    pos_flat = positions.reshape(-1)
    tbl_flat = jnp.pad(embedding.reshape(-1), (0, TBL_PAD - TBL))
    out = _lookup(pos_flat, tbl_flat)
    return out.reshape(ROWS, COLS, DEPTH)


# speedup vs baseline: 57.0640x; 11.0985x over previous
import functools

import jax
import jax.numpy as jnp
from jax import lax
from jax.experimental import pallas as pl
from jax.experimental.pallas import tpu as pltpu
from jax.experimental.pallas import tpu_sc as plsc

MAX_POS = 2048
DEPTH = 4
ROWS, COLS = 16384, 200
N = ROWS * COLS
NC, NS, L = 2, 16, 16
NW = NC * NS
RT = ROWS // 128
TASKS = COLS * 4
PER_W = TASKS // NW
TQ = RT // 4
TBL = (MAX_POS + 1) * DEPTH
TBL_PAD = 8208

_mesh = plsc.VectorSubcoreMesh(core_axis_name="c", subcore_axis_name="s")


def _task_slices(pos_hbm, out_hbm, t):
    c = t // 4
    rt0 = (t % 4) * TQ
    src = pos_hbm.at[c // 8, pl.ds(rt0, TQ), c % 8, :]
    dst = out_hbm.at[pl.ds(t * TQ * DEPTH, TQ * DEPTH), :]
    return src, dst


@functools.partial(
    pl.kernel,
    out_type=jax.ShapeDtypeStruct((TASKS * TQ * DEPTH, 128), jnp.float32),
    mesh=_mesh,
    scratch_types=[
        pltpu.VMEM((TBL_PAD,), jnp.float32),
        pltpu.VMEM((TQ, 128), jnp.int32),
        pltpu.VMEM((TQ, 128), jnp.int32),
        pltpu.VMEM((TQ * DEPTH, 128), jnp.float32),
        pltpu.VMEM((TQ * DEPTH, 128), jnp.float32),
        pltpu.SemaphoreType.DMA,
        pltpu.SemaphoreType.DMA,
        pltpu.SemaphoreType.DMA,
        pltpu.SemaphoreType.DMA,
    ],
    compiler_params=pltpu.CompilerParams(needs_layout_passes=False),
)
def _lookup(pos_hbm, tbl_hbm, out_hbm,
            tbl_v, idx0, idx1, out0, out1, sin0, sin1, sout0, sout1):
    wid = lax.axis_index("s") * NC + lax.axis_index("c")
    t0 = wid * PER_W

    pltpu.sync_copy(tbl_hbm, tbl_v)

    bufs = ((idx0, out0, sin0, sout0), (idx1, out1, sin1, sout1))

    for s in range(2):
        src, _ = _task_slices(pos_hbm, out_hbm, t0 + s)
        pltpu.async_copy(src, bufs[s][0], bufs[s][2])

    def compute(idxb, outb):
        def row_body(j, _):
            for i in range(8):
                v = idxb[j, pl.ds(i * L, L)]
                f = jnp.minimum(v, MAX_POS) * DEPTH
                for k in range(DEPTH):
                    gk = plsc.load_gather(tbl_v, [f + k])
                    outb[j * DEPTH + k, pl.ds(i * L, L)] = gk
            return 0

        lax.fori_loop(0, TQ, row_body, 0)

    for p in range(PER_W):
        idxb, outb, sin, sout = bufs[p % 2]
        t = t0 + p
        src, dst = _task_slices(pos_hbm, out_hbm, t)
        pltpu.make_async_copy(src, idxb, sin).wait()
        if p >= 2:
            psrc, pdst = _task_slices(pos_hbm, out_hbm, t - 2)
            pltpu.make_async_copy(outb, pdst, sout).wait()
        compute(idxb, outb)
        pltpu.async_copy(outb, dst, sout)
        if p + 2 < PER_W:
            nsrc, _ = _task_slices(pos_hbm, out_hbm, t + 2)
            pltpu.async_copy(nsrc, idxb, sin)

    for p in range(max(PER_W - 2, 0), PER_W):
        idxb, outb, sin, sout = bufs[p % 2]
        _, dst = _task_slices(pos_hbm, out_hbm, t0 + p)
        pltpu.make_async_copy(outb, dst, sout).wait()


def kernel(positions, embedding):
    pos4d = (positions.T.reshape(COLS // 8, 8, RT, 128)
             .transpose(0, 2, 1, 3))
    tbl_flat = jnp.pad(embedding.reshape(-1), (0, TBL_PAD - TBL))
    out = _lookup(pos4d, tbl_flat)
    out = (out.reshape(COLS, 4, TQ, DEPTH, 128)
           .transpose(1, 2, 4, 0, 3)
           .reshape(ROWS, COLS, DEPTH))
    return out


# speedup vs baseline: 142.1850x; 2.4917x over previous
import functools

import jax
import jax.numpy as jnp
from jax import lax
from jax.experimental import pallas as pl
from jax.experimental.pallas import tpu as pltpu
from jax.experimental.pallas import tpu_sc as plsc

MAX_POS = 2048
DEPTH = 4
ROWS, COLS = 16384, 200
N = ROWS * COLS
NC, NS, L = 2, 16, 16
NW = NC * NS
RT = ROWS // 128
TASKS = COLS * 4
PER_W = TASKS // NW
TQ = RT // 4
TBL = (MAX_POS + 1) * DEPTH
TBL_PAD = 8208

_mesh = plsc.VectorSubcoreMesh(core_axis_name="c", subcore_axis_name="s")


def _task_slices(pos_hbm, out_hbm, t):
    c = t // 4
    rt0 = (t % 4) * TQ
    src = pos_hbm.at[c // 8, pl.ds(rt0, TQ), c % 8, :]
    dst = out_hbm.at[pl.ds(t * TQ * DEPTH, TQ * DEPTH), :]
    return src, dst


@functools.partial(
    pl.kernel,
    out_type=jax.ShapeDtypeStruct((TASKS * TQ * DEPTH, 128), jnp.float32),
    mesh=_mesh,
    scratch_types=[
        pltpu.VMEM((TBL_PAD,), jnp.float32),
        pltpu.VMEM((TQ, 128), jnp.int32),
        pltpu.VMEM((TQ, 128), jnp.int32),
        pltpu.VMEM((TQ * DEPTH, 128), jnp.float32),
        pltpu.VMEM((TQ * DEPTH, 128), jnp.float32),
        pltpu.SemaphoreType.DMA,
        pltpu.SemaphoreType.DMA,
        pltpu.SemaphoreType.DMA,
        pltpu.SemaphoreType.DMA,
    ],
    compiler_params=pltpu.CompilerParams(needs_layout_passes=False),
)
def _lookup(pos_hbm, tbl_hbm, out_hbm,
            tbl_v, idx0, idx1, out0, out1, sin0, sin1, sout0, sout1):
    wid = lax.axis_index("s") * NC + lax.axis_index("c")
    t0 = wid * PER_W

    pltpu.sync_copy(tbl_hbm, tbl_v)

    bufs = ((idx0, out0, sin0, sout0), (idx1, out1, sin1, sout1))
    t_last = t0 + PER_W - 1

    for s in range(2):
        src, _ = _task_slices(pos_hbm, out_hbm, t0 + s)
        pltpu.async_copy(src, bufs[s][0], bufs[s][2])

    def compute(idxb, outb):
        @plsc.parallel_loop(0, TQ, 1, unroll=2)
        def row_body(j):
            for i in range(8):
                v = idxb[j, pl.ds(i * L, L)]
                f = jnp.minimum(v, MAX_POS) * DEPTH
                for k in range(DEPTH):
                    gk = plsc.load_gather(tbl_v, [f + k])
                    outb[j * DEPTH + k, pl.ds(i * L, L)] = gk

    def pair_body(q, _):
        t = t0 + 2 * q
        for s in range(2):
            idxb, outb, sin, sout = bufs[s]
            src, dst = _task_slices(pos_hbm, out_hbm, t + s)
            pltpu.make_async_copy(src, idxb, sin).wait()
            compute(idxb, outb)
            pltpu.async_copy(outb, dst, sout)
            nsrc, _ = _task_slices(
                pos_hbm, out_hbm, jnp.minimum(t + s + 2, t_last))
            pltpu.async_copy(nsrc, idxb, sin)
        for s in range(2):
            idxb, outb, sin, sout = bufs[s]
            _, dst = _task_slices(pos_hbm, out_hbm, t + s)
            pltpu.make_async_copy(outb, dst, sout).wait()
        return 0

    lax.fori_loop(0, (PER_W - 1) // 2, pair_body, 0)

    idxb, outb, sin, sout = bufs[0]
    src, dst = _task_slices(pos_hbm, out_hbm, t_last)
    pltpu.make_async_copy(src, idxb, sin).wait()
    compute(idxb, outb)
    pltpu.async_copy(outb, dst, sout)
    pltpu.make_async_copy(outb, dst, sout).wait()
    src1, _ = _task_slices(pos_hbm, out_hbm, t_last)
    pltpu.make_async_copy(src1, bufs[1][0], bufs[1][2]).wait()


def kernel(positions, embedding):
    pos4d = (positions.T.reshape(COLS // 8, 8, RT, 128)
             .transpose(0, 2, 1, 3))
    tbl_flat = jnp.pad(embedding.reshape(-1), (0, TBL_PAD - TBL))
    out = _lookup(pos4d, tbl_flat)
    out = (out.reshape(COLS, 4, TQ, DEPTH, 128)
           .transpose(1, 2, 4, 0, 3)
           .reshape(ROWS, COLS, DEPTH))
    return out


# speedup vs baseline: 149.3896x; 1.0507x over previous
import functools

import jax
import jax.numpy as jnp
from jax import lax
from jax.experimental import pallas as pl
from jax.experimental.pallas import tpu as pltpu
from jax.experimental.pallas import tpu_sc as plsc

MAX_POS = 2048
DEPTH = 4
ROWS, COLS = 16384, 200
N = ROWS * COLS
NC, NS, L = 2, 16, 16
NW = NC * NS
RT = ROWS // 128
TASKS = COLS * 4
PER_W = TASKS // NW
TQ = RT // 4
TBL = (MAX_POS + 1) * DEPTH
TBL_PAD = 8208

_mesh = plsc.VectorSubcoreMesh(core_axis_name="c", subcore_axis_name="s")


def _task_slices(pos_hbm, out_hbm, t):
    c = t // 4
    rt0 = (t % 4) * TQ
    src = pos_hbm.at[c // 8, pl.ds(rt0, TQ), c % 8, :]
    dst = out_hbm.at[pl.ds(t * TQ * DEPTH, TQ * DEPTH), :]
    return src, dst


@functools.partial(
    pl.kernel,
    out_type=jax.ShapeDtypeStruct((TASKS * TQ * DEPTH, 128), jnp.float32),
    mesh=_mesh,
    scratch_types=[
        pltpu.VMEM((TBL_PAD,), jnp.float32),
        pltpu.VMEM((TQ, 128), jnp.int32),
        pltpu.VMEM((TQ, 128), jnp.int32),
        pltpu.VMEM((TQ * DEPTH, 128), jnp.float32),
        pltpu.VMEM((TQ * DEPTH, 128), jnp.float32),
        pltpu.SemaphoreType.DMA,
        pltpu.SemaphoreType.DMA,
        pltpu.SemaphoreType.DMA,
        pltpu.SemaphoreType.DMA,
    ],
    compiler_params=pltpu.CompilerParams(needs_layout_passes=False),
)
def _lookup(pos_hbm, tbl_hbm, out_hbm,
            tbl_v, idx0, idx1, out0, out1, sin0, sin1, sout0, sout1):
    wid = lax.axis_index("s") * NC + lax.axis_index("c")
    t0 = wid * PER_W

    pltpu.sync_copy(tbl_hbm, tbl_v)

    bufs = ((idx0, out0, sin0, sout0), (idx1, out1, sin1, sout1))
    t_last = t0 + PER_W - 1

    for s in range(2):
        src, _ = _task_slices(pos_hbm, out_hbm, t0 + s)
        pltpu.async_copy(src, bufs[s][0], bufs[s][2])

    def compute(idxb, outb):
        @plsc.parallel_loop(0, TQ, 1, unroll=4)
        def row_body(j):
            for i in range(8):
                v = idxb[j, pl.ds(i * L, L)]
                f = jnp.minimum(v, MAX_POS) * DEPTH
                for k in range(DEPTH):
                    gk = plsc.load_gather(tbl_v, [f + k])
                    outb[j * DEPTH + k, pl.ds(i * L, L)] = gk

    def pair_body(q, _):
        t = t0 + 2 * q
        for s in range(2):
            idxb, outb, sin, sout = bufs[s]
            src, dst = _task_slices(pos_hbm, out_hbm, t + s)
            pltpu.make_async_copy(src, idxb, sin).wait()

            @pl.when(q > 0)
            def _():
                pltpu.make_async_copy(outb, dst, sout).wait()

            compute(idxb, outb)
            pltpu.async_copy(outb, dst, sout)
            nsrc, _ = _task_slices(
                pos_hbm, out_hbm, jnp.minimum(t + s + 2, t_last))
            pltpu.async_copy(nsrc, idxb, sin)
        return 0

    lax.fori_loop(0, (PER_W - 1) // 2, pair_body, 0)

    idxb, outb, sin, sout = bufs[0]
    src, dst = _task_slices(pos_hbm, out_hbm, t_last)
    pltpu.make_async_copy(src, idxb, sin).wait()
    pltpu.make_async_copy(outb, dst, sout).wait()
    compute(idxb, outb)
    pltpu.async_copy(outb, dst, sout)
    pltpu.make_async_copy(outb, dst, sout).wait()
    pltpu.make_async_copy(bufs[1][1], dst, bufs[1][3]).wait()
    src1, _ = _task_slices(pos_hbm, out_hbm, t_last)
    pltpu.make_async_copy(src1, bufs[1][0], bufs[1][2]).wait()


def kernel(positions, embedding):
    pos4d = (positions.T.reshape(COLS // 8, 8, RT, 128)
             .transpose(0, 2, 1, 3))
    tbl_flat = jnp.pad(embedding.reshape(-1), (0, TBL_PAD - TBL))
    out = _lookup(pos4d, tbl_flat)
    out = (out.reshape(COLS, 4, TQ, DEPTH, 128)
           .transpose(1, 2, 4, 0, 3)
           .reshape(ROWS, COLS, DEPTH))
    return out


# speedup vs baseline: 179.3961x; 1.2009x over previous
import functools

import jax
import jax.numpy as jnp
from jax import lax
from jax.experimental import pallas as pl
from jax.experimental.pallas import tpu as pltpu
from jax.experimental.pallas import tpu_sc as plsc

MAX_POS = 2048
DEPTH = 4
ROWS, COLS = 16384, 200
N = ROWS * COLS
NC, NS, L = 2, 16, 16
NW = NC * NS
RT = ROWS // 128
TASKS = COLS * 4
PER_W = TASKS // NW
TQ = RT // 4
TBL_ROWS = MAX_POS + 1
PLANE = 2052
TBL_PAD = DEPTH * PLANE

_mesh = plsc.VectorSubcoreMesh(core_axis_name="c", subcore_axis_name="s")


def _task_slices(pos_hbm, out_hbm, t):
    c = t // 4
    rt0 = (t % 4) * TQ
    src = pos_hbm.at[c // 8, pl.ds(rt0, TQ), c % 8, :]
    dst = out_hbm.at[pl.ds(t * TQ * DEPTH, TQ * DEPTH), :]
    return src, dst


@functools.partial(
    pl.kernel,
    out_type=jax.ShapeDtypeStruct((TASKS * TQ * DEPTH, 128), jnp.float32),
    mesh=_mesh,
    scratch_types=[
        pltpu.VMEM((TBL_PAD,), jnp.float32),
        pltpu.VMEM((TQ, 128), jnp.int32),
        pltpu.VMEM((TQ, 128), jnp.int32),
        pltpu.VMEM((TQ * DEPTH, 128), jnp.float32),
        pltpu.VMEM((TQ * DEPTH, 128), jnp.float32),
        pltpu.SemaphoreType.DMA,
        pltpu.SemaphoreType.DMA,
        pltpu.SemaphoreType.DMA,
        pltpu.SemaphoreType.DMA,
    ],
    compiler_params=pltpu.CompilerParams(needs_layout_passes=False),
)
def _lookup(pos_hbm, tbl_hbm, out_hbm,
            tbl_v, idx0, idx1, out0, out1, sin0, sin1, sout0, sout1):
    wid = lax.axis_index("s") * NC + lax.axis_index("c")
    t0 = wid * PER_W

    pltpu.sync_copy(tbl_hbm, tbl_v)

    bufs = ((idx0, out0, sin0, sout0), (idx1, out1, sin1, sout1))
    t_last = t0 + PER_W - 1

    for s in range(2):
        src, _ = _task_slices(pos_hbm, out_hbm, t0 + s)
        pltpu.async_copy(src, bufs[s][0], bufs[s][2])

    def compute(idxb, outb):
        @plsc.parallel_loop(0, TQ, 1, unroll=4)
        def row_body(j):
            for i in range(8):
                v = idxb[j, pl.ds(i * L, L)]
                f = jnp.minimum(v, MAX_POS)
                for k in range(DEPTH):
                    gk = plsc.load_gather(tbl_v, [f + k * PLANE])
                    outb[j * DEPTH + k, pl.ds(i * L, L)] = gk

    def pair_body(q, _):
        t = t0 + 2 * q
        for s in range(2):
            idxb, outb, sin, sout = bufs[s]
            src, dst = _task_slices(pos_hbm, out_hbm, t + s)
            pltpu.make_async_copy(src, idxb, sin).wait()

            @pl.when(q > 0)
            def _():
                pltpu.make_async_copy(outb, dst, sout).wait()

            compute(idxb, outb)
            pltpu.async_copy(outb, dst, sout)
            nsrc, _ = _task_slices(
                pos_hbm, out_hbm, jnp.minimum(t + s + 2, t_last))
            pltpu.async_copy(nsrc, idxb, sin)
        return 0

    lax.fori_loop(0, (PER_W - 1) // 2, pair_body, 0)

    idxb, outb, sin, sout = bufs[0]
    src, dst = _task_slices(pos_hbm, out_hbm, t_last)
    pltpu.make_async_copy(src, idxb, sin).wait()
    pltpu.make_async_copy(outb, dst, sout).wait()
    compute(idxb, outb)
    pltpu.async_copy(outb, dst, sout)
    pltpu.make_async_copy(outb, dst, sout).wait()
    pltpu.make_async_copy(bufs[1][1], dst, bufs[1][3]).wait()
    src1, _ = _task_slices(pos_hbm, out_hbm, t_last)
    pltpu.make_async_copy(src1, bufs[1][0], bufs[1][2]).wait()


def kernel(positions, embedding):
    pos4d = (positions.T.reshape(COLS // 8, 8, RT, 128)
             .transpose(0, 2, 1, 3))
    tbl_flat = jnp.pad(embedding.T, ((0, 0), (0, PLANE - TBL_ROWS))).reshape(-1)
    out = _lookup(pos4d, tbl_flat)
    out = (out.reshape(COLS, 4, TQ, DEPTH, 128)
           .transpose(1, 2, 4, 0, 3)
           .reshape(ROWS, COLS, DEPTH))
    return out
